# baseline (speedup 1.0000x reference)
import functools

import jax
import jax.numpy as jnp
from jax import lax
from jax.experimental import pallas as pl
from jax.experimental.pallas import tpu as pltpu
from jax.experimental.pallas import tpu_sc as plsc

B = 128
HW = 49
C = 512
ROW_PAD = 64
THRESHOLD = 0.5
L = 16

_INFO = plsc.get_sparse_core_info()
NC = _INFO.num_cores
NS = _INFO.num_subcores
NW = NC * NS
SPT = B // NW


_GDN = lax.GatherDimensionNumbers(
    offset_dims=(), collapsed_slice_dims=(0,), start_index_map=(0,)
)


def _lane_perm(vec, idx):
    return lax.gather(
        vec, idx[:, None], _GDN, (1,),
        mode=lax.GatherScatterMode.PROMISE_IN_BOUNDS,
    )


def _sc_body(bA_hbm, interm_hbm, aout_hbm, bA_v, rows_v, sem):
    wid = lax.axis_index("s") * NC + lax.axis_index("c")
    base = wid * SPT
    pltpu.sync_copy(bA_hbm.at[pl.ds(base, SPT)], bA_v)
    lanes = lax.iota(jnp.int32, L)
    copies = []
    for s in range(SPT):
        def amax_body(j, carry, s=s):
            bv, bi = carry
            v = bA_v[s, pl.ds(j * L, L)]
            idx = j * L + lanes
            take = v > bv
            return (jnp.where(take, v, bv), jnp.where(take, idx, bi))

        bv0 = bA_v[s, pl.ds(0, L)]
        bv, bi = lax.fori_loop(1, C // L, amax_body, (bv0, lanes))
        for sh in (8, 4, 2, 1):
            perm = lanes ^ sh
            pv = _lane_perm(bv, perm)
            pi = _lane_perm(bi, perm)
            take = (pv > bv) | ((pv == bv) & (pi < bi))
            bv = jnp.where(take, pv, bv)
            bi = jnp.where(take, pi, bi)
        sample_base = (base + s) * (HW * C) + bi
        for t in range(ROW_PAD // L):
            p = jnp.minimum(t * L + lanes, HW - 1)
            flat = sample_base + p * C
            copies.append(
                pltpu.async_copy(
                    interm_hbm.at[flat], rows_v.at[s, pl.ds(t * L, L)], sem
                )
            )
    for cp in copies:
        cp.wait()
    pltpu.sync_copy(rows_v, aout_hbm.at[pl.ds(base, SPT)])


_sc_gather = functools.partial(
    pl.kernel,
    out_type=jax.ShapeDtypeStruct((B, ROW_PAD), jnp.float32),
    mesh=plsc.VectorSubcoreMesh(core_axis_name="c", subcore_axis_name="s"),
    scratch_types=[
        pltpu.VMEM((SPT, C), jnp.float32),
        pltpu.VMEM((SPT, ROW_PAD), jnp.float32),
        pltpu.SemaphoreType.DMA,
    ],
)(_sc_body)


R_BLK = 784


def _tc_body(v_ref, a_ref, o_ref):
    a = a_ref[...]
    tmp = jnp.where(a > THRESHOLD, a, 0.0)
    o_ref[...] = v_ref[...] - tmp


def kernel(vgg_end, interm, branchA_end):
    aout = _sc_gather(branchA_end, interm.reshape(-1))
    amask = aout[:, :HW].reshape(B * HW, 1)
    vgg2 = vgg_end.reshape(B * HW, C)
    out2 = pl.pallas_call(
        _tc_body,
        grid=(B * HW // R_BLK,),
        in_specs=[
            pl.BlockSpec((R_BLK, C), lambda i: (i, 0)),
            pl.BlockSpec((R_BLK, 1), lambda i: (i, 0)),
        ],
        out_specs=pl.BlockSpec((R_BLK, C), lambda i: (i, 0)),
        out_shape=jax.ShapeDtypeStruct((B * HW, C), jnp.float32),
    )(vgg2, amask)
    return out2.reshape(B, 7, 7, C)
